# Initial kernel scaffold; baseline (speedup 1.0000x reference)
import functools

import jax
import jax.numpy as jnp
from jax import lax
from jax.experimental import pallas as pl
from jax.experimental.pallas import tpu as pltpu
from jax.experimental.pallas import tpu_sc as plsc

N = 10000
D = 128
N_PAD = 10240
NW = 32
CH = 128
NR16 = N_PAD // 16
ROWS_PER_TILE = N_PAD // 16
DROWS_PER_TILE = NR16 // 16
NRC = NR16 // CH

_mesh = plsc.VectorSubcoreMesh(core_axis_name="c", subcore_axis_name="s")


def _make_sc_agg(n_chunks, with_deg):
  out_type = [jax.ShapeDtypeStruct((2 * N_PAD, D), jnp.float32)]
  if with_deg:
    out_type.append(jax.ShapeDtypeStruct((2 * NR16, 16), jnp.float32))
  scratch = [
      pltpu.VMEM((n_chunks, CH), jnp.int32),
      pltpu.VMEM((n_chunks, CH), jnp.int32),
      pltpu.VMEM((CH, D), jnp.float32),
      pltpu.VMEM_SHARED((N_PAD, D), jnp.float32),
      pltpu.SemaphoreType.DMA,
  ]
  if with_deg:
    scratch += [
        pltpu.VMEM((NR16, 16), jnp.float32),
        pltpu.VMEM((NRC, CH), jnp.int32),
        pltpu.VMEM_SHARED((NR16, 16), jnp.float32),
    ]

  def body(h_hbm, srcr_hbm, dstr_hbm, *refs):
    if with_deg:
      (acc_hbm, deg_hbm, src_v, dst_v, rows_v, acc_sh, sem,
       deg_v, ridx_v, deg_sh) = refs
    else:
      acc_hbm, src_v, dst_v, rows_v, acc_sh, sem = refs
    cid = lax.axis_index("c")
    sid = lax.axis_index("s")
    wid = cid * 16 + sid

    zero16 = jnp.zeros((16,), jnp.float32)

    @pl.loop(0, CH)
    def _(r):
      @pl.loop(0, D, step=16)
      def _(c):
        rows_v[r, pl.ds(c, 16)] = zero16

    for k in range(ROWS_PER_TILE // CH):
      pltpu.sync_copy(rows_v, acc_sh.at[pl.ds(sid * ROWS_PER_TILE + k * CH, CH)])

    if with_deg:
      @pl.loop(0, NR16)
      def _(r):
        deg_v[r, :] = zero16
      pltpu.sync_copy(deg_v.at[pl.ds(0, DROWS_PER_TILE)],
                      deg_sh.at[pl.ds(sid * DROWS_PER_TILE, DROWS_PER_TILE)])
      iota16 = lax.iota(jnp.int32, 16)
      for c in range(NRC):
        for j in range(CH // 16):
          ridx_v[c, pl.ds(j * 16, 16)] = iota16 + (c * CH + j * 16)

    plsc.subcore_barrier()

    pltpu.sync_copy(srcr_hbm.at[wid], src_v)
    pltpu.sync_copy(dstr_hbm.at[wid], dst_v)

    ones16 = jnp.ones((16,), jnp.float32)

    @pl.loop(0, n_chunks)
    def _(i):
      pltpu.async_copy(h_hbm.at[src_v.at[i]], rows_v, sem).wait()
      pltpu.sync_copy(rows_v, acc_sh.at[dst_v.at[i]], add=True)
      if with_deg:
        for j in range(CH // 16):
          idx = dst_v[i, pl.ds(j * 16, 16)]
          plsc.addupdate_scatter(
              deg_v,
              [lax.shift_right_logical(idx, 4), lax.bitwise_and(idx, 15)],
              ones16)

    if with_deg:
      for c in range(NRC):
        pltpu.sync_copy(deg_v.at[pl.ds(c * CH, CH)],
                        deg_sh.at[ridx_v.at[c]], add=True)

    plsc.subcore_barrier()

    base = sid * ROWS_PER_TILE
    pltpu.sync_copy(acc_sh.at[pl.ds(base, ROWS_PER_TILE)],
                    acc_hbm.at[pl.ds(cid * N_PAD + base, ROWS_PER_TILE)])
    if with_deg:
      dbase = sid * DROWS_PER_TILE
      pltpu.sync_copy(deg_sh.at[pl.ds(dbase, DROWS_PER_TILE)],
                      deg_hbm.at[pl.ds(cid * NR16 + dbase, DROWS_PER_TILE)])

  return pl.kernel(body, out_type=out_type, mesh=_mesh, scratch_types=scratch)


_DOT = functools.partial(
    lax.dot_general,
    dimension_numbers=(((1,), (0,)), ((), ())),
    preferred_element_type=jnp.float32,
    precision=lax.Precision.HIGHEST)


def _k_in_body(x_ref, w_ref, b_ref, o_ref):
  o_ref[...] = jnp.tanh(_DOT(x_ref[...], w_ref[...]) + b_ref[...])


def _k_layer_body(h_ref, a0_ref, a1_ref, d0_ref, d1_ref, ws_ref, b_ref,
                  wn_ref, o_ref):
  deg = jnp.maximum(d0_ref[...] + d1_ref[...], 1.0)
  mean = (a0_ref[...] + a1_ref[...]) / deg
  acc = _DOT(h_ref[...], ws_ref[...]) + _DOT(mean, wn_ref[...])
  o_ref[...] = jnp.maximum(acc + b_ref[...], 0.0)


_BLK = 1024
_GRID = N_PAD // _BLK
_row_spec = pl.BlockSpec((_BLK, D), lambda i: (i, 0))
_w_spec = pl.BlockSpec((D, D), lambda i: (0, 0))
_b_spec = pl.BlockSpec((1, D), lambda i: (0, 0))
_out_sds = jax.ShapeDtypeStruct((N_PAD, D), jnp.float32)

_k_in = pl.pallas_call(
    _k_in_body,
    grid=(_GRID,),
    in_specs=[_row_spec, _w_spec, _b_spec],
    out_specs=_row_spec,
    out_shape=_out_sds)

_k_layer = pl.pallas_call(
    _k_layer_body,
    grid=(_GRID,),
    in_specs=[
        _row_spec,
        pl.BlockSpec((_BLK, D), lambda i: (i, 0)),
        pl.BlockSpec((_BLK, D), lambda i: (i + _GRID, 0)),
        pl.BlockSpec((_BLK, 1), lambda i: (i, 0)),
        pl.BlockSpec((_BLK, 1), lambda i: (i + _GRID, 0)),
        _w_spec, _b_spec, _w_spec,
    ],
    out_specs=_row_spec,
    out_shape=_out_sds)


def kernel(x, edge_index, W_in, b_in, W_self1, b_self1, W_neigh1,
           W_self2, b_self2, W_neigh2):
  E = edge_index.shape[1]
  n_chunks = -(-E // (NW * CH))
  e_pad = NW * CH * n_chunks - E

  xp = jnp.zeros((N_PAD, D), jnp.float32).at[:N].set(x)
  src = edge_index[0]
  dst = edge_index[1]
  if e_pad:
    ar = jnp.arange(e_pad, dtype=jnp.int32)
    src = jnp.concatenate([src, ar % N])
    dst = jnp.concatenate([dst, N + ar % (N_PAD - N)])
  srcr = src.reshape(NW, n_chunks, CH)
  dstr = dst.reshape(NW, n_chunks, CH)

  sc_agg_deg = _make_sc_agg(n_chunks, with_deg=True)
  sc_agg = _make_sc_agg(n_chunks, with_deg=False)

  b_in2 = b_in.reshape(1, D)
  b1 = b_self1.reshape(1, D)
  b2 = b_self2.reshape(1, D)

  h0 = _k_in(xp, W_in, b_in2)
  acc1, deg = sc_agg_deg(h0, srcr, dstr)
  degf = deg.reshape(2 * N_PAD, 1)
  h1 = _k_layer(h0, acc1, acc1, degf, degf, W_self1, b1, W_neigh1)
  acc2 = sc_agg(h1, srcr, dstr)
  if isinstance(acc2, (list, tuple)):
    acc2 = acc2[0]
  h2 = _k_layer(h1, acc2, acc2, degf, degf, W_self2, b2, W_neigh2)
  return h2[:N]


# speedup vs baseline: 8.5738x; 8.5738x over previous
import dataclasses
import functools

import jax
import jax.numpy as jnp
from jax import lax
from jax.experimental import pallas as pl
from jax.experimental.pallas import tpu as pltpu
from jax.experimental.pallas import tpu_sc as plsc

N = 10000
D = 128
N_PAD = 10240
NW = 32
CH = 128
NDR = N_PAD // D
ROWS_PER_TILE = N_PAD // 16
DROWS_PER_TILE = NDR // 16

_mesh = plsc.VectorSubcoreMesh(core_axis_name="c", subcore_axis_name="s")

_sc_params = pltpu.CompilerParams()
if "needs_layout_passes" in pltpu.CompilerParams.__dataclass_fields__:
  _sc_params = dataclasses.replace(_sc_params, needs_layout_passes=False)


def _make_sc_agg(n_chunks):
  out_type = [jax.ShapeDtypeStruct((2 * N_PAD, D), jnp.float32),
              jax.ShapeDtypeStruct((2 * NDR, D), jnp.float32)]
  scratch = [
      pltpu.VMEM((n_chunks, CH), jnp.int32),
      pltpu.VMEM((n_chunks, CH), jnp.int32),
      pltpu.VMEM((CH, D), jnp.float32),
      pltpu.VMEM((NDR, D), jnp.float32),
      pltpu.VMEM((1, NDR), jnp.int32),
      pltpu.VMEM_SHARED((N_PAD, D), jnp.float32),
      pltpu.VMEM_SHARED((NDR, D), jnp.float32),
      pltpu.SemaphoreType.DMA,
  ]

  def body(h_hbm, srcr_hbm, dstr_hbm, acc_hbm, deg_hbm,
           src_v, dst_v, rows_v, deg_v, ridx_v, acc_sh, deg_sh, sem):
    cid = lax.axis_index("c")
    sid = lax.axis_index("s")
    wid = cid * 16 + sid

    zero16 = jnp.zeros((16,), jnp.float32)

    @pl.loop(0, CH)
    def _(r):
      @pl.loop(0, D, step=16)
      def _(c):
        rows_v[r, pl.ds(c, 16)] = zero16

    for k in range(ROWS_PER_TILE // CH):
      pltpu.sync_copy(rows_v, acc_sh.at[pl.ds(sid * ROWS_PER_TILE + k * CH, CH)])

    @pl.loop(0, NDR)
    def _(r):
      @pl.loop(0, D, step=16)
      def _(c):
        deg_v[r, pl.ds(c, 16)] = zero16
    @pl.when(sid == 0)
    def _():
      pltpu.sync_copy(deg_v, deg_sh)
    iota16 = lax.iota(jnp.int32, 16)
    for j in range(NDR // 16):
      ridx_v[0, pl.ds(j * 16, 16)] = iota16 + j * 16

    plsc.subcore_barrier()

    pltpu.sync_copy(srcr_hbm.at[wid], src_v)
    pltpu.sync_copy(dstr_hbm.at[wid], dst_v)

    ones16 = jnp.ones((16,), jnp.float32)

    @pl.loop(0, n_chunks)
    def _(i):
      pltpu.async_copy(h_hbm.at[src_v.at[i]], rows_v, sem).wait()
      pltpu.sync_copy(rows_v, acc_sh.at[dst_v.at[i]], add=True)
      for j in range(CH // 16):
        idx = dst_v[i, pl.ds(j * 16, 16)]
        plsc.addupdate_scatter(
            deg_v,
            [lax.shift_right_logical(idx, 7), lax.bitwise_and(idx, 127)],
            ones16)

    pltpu.sync_copy(deg_v, deg_sh.at[ridx_v.at[0]], add=True)

    plsc.subcore_barrier()

    base = sid * ROWS_PER_TILE
    pltpu.sync_copy(acc_sh.at[pl.ds(base, ROWS_PER_TILE)],
                    acc_hbm.at[pl.ds(cid * N_PAD + base, ROWS_PER_TILE)])
    @pl.when(sid == 0)
    def _():
      pltpu.sync_copy(deg_sh, deg_hbm.at[pl.ds(cid * NDR, NDR)])

  return pl.kernel(body, out_type=out_type, mesh=_mesh, scratch_types=scratch,
                   compiler_params=_sc_params)


_DOT = functools.partial(
    lax.dot_general,
    dimension_numbers=(((1,), (0,)), ((), ())),
    preferred_element_type=jnp.float32,
    precision=lax.Precision.HIGHEST)


def _k_in_body(x_ref, w_ref, b_ref, o_ref):
  o_ref[...] = jnp.tanh(_DOT(x_ref[...], w_ref[...]) + b_ref[...])


def _k_layer_body(h_ref, a0_ref, a1_ref, d0_ref, d1_ref, ws_ref, b_ref,
                  wn_ref, o_ref):
  deg = jnp.maximum(d0_ref[...] + d1_ref[...], 1.0)
  mean = (a0_ref[...] + a1_ref[...]) / deg
  acc = _DOT(h_ref[...], ws_ref[...]) + _DOT(mean, wn_ref[...])
  o_ref[...] = jnp.maximum(acc + b_ref[...], 0.0)


_BLK = 1024
_GRID = N_PAD // _BLK
_row_spec = pl.BlockSpec((_BLK, D), lambda i: (i, 0))
_w_spec = pl.BlockSpec((D, D), lambda i: (0, 0))
_b_spec = pl.BlockSpec((1, D), lambda i: (0, 0))
_out_sds = jax.ShapeDtypeStruct((N_PAD, D), jnp.float32)

_k_in = pl.pallas_call(
    _k_in_body,
    grid=(_GRID,),
    in_specs=[_row_spec, _w_spec, _b_spec],
    out_specs=_row_spec,
    out_shape=_out_sds)

_k_layer = pl.pallas_call(
    _k_layer_body,
    grid=(_GRID,),
    in_specs=[
        _row_spec,
        pl.BlockSpec((_BLK, D), lambda i: (i, 0)),
        pl.BlockSpec((_BLK, D), lambda i: (i + _GRID, 0)),
        pl.BlockSpec((_BLK, 1), lambda i: (i, 0)),
        pl.BlockSpec((_BLK, 1), lambda i: (i + _GRID, 0)),
        _w_spec, _b_spec, _w_spec,
    ],
    out_specs=_row_spec,
    out_shape=_out_sds)


def kernel(x, edge_index, W_in, b_in, W_self1, b_self1, W_neigh1,
           W_self2, b_self2, W_neigh2):
  E = edge_index.shape[1]
  n_chunks = -(-E // (NW * CH))
  e_pad = NW * CH * n_chunks - E

  xp = jnp.zeros((N_PAD, D), jnp.float32).at[:N].set(x)
  src = edge_index[0]
  dst = edge_index[1]
  if e_pad:
    ar = jnp.arange(e_pad, dtype=jnp.int32)
    src = jnp.concatenate([src, ar % N])
    dst = jnp.concatenate([dst, N + ar % (N_PAD - N)])
  srcr = src.reshape(NW, n_chunks, CH)
  dstr = dst.reshape(NW, n_chunks, CH)

  sc_agg = _make_sc_agg(n_chunks)

  b_in2 = b_in.reshape(1, D)
  b1 = b_self1.reshape(1, D)
  b2 = b_self2.reshape(1, D)

  h0 = _k_in(xp, W_in, b_in2)
  acc1, deg = sc_agg(h0, srcr, dstr)
  degf = deg.reshape(2 * N_PAD, 1)
  h1 = _k_layer(h0, acc1, acc1, degf, degf, W_self1, b1, W_neigh1)
  acc2, _deg2 = sc_agg(h1, srcr, dstr)
  h2 = _k_layer(h1, acc2, acc2, degf, degf, W_self2, b2, W_neigh2)
  return h2[:N]


# speedup vs baseline: 9.1072x; 1.0622x over previous
import dataclasses
import functools

import jax
import jax.numpy as jnp
from jax import lax
from jax.experimental import pallas as pl
from jax.experimental.pallas import tpu as pltpu
from jax.experimental.pallas import tpu_sc as plsc

N = 10000
D = 128
DH = D // 2
N_PAD = 10240
NW = 32
CH = 128
NBUF = 4
NDR = N_PAD // D
ROWS_PER_TILE = N_PAD // 16

_mesh = plsc.VectorSubcoreMesh(core_axis_name="c", subcore_axis_name="s")

_sc_params = pltpu.CompilerParams()
if "needs_layout_passes" in pltpu.CompilerParams.__dataclass_fields__:
  _sc_params = dataclasses.replace(_sc_params, needs_layout_passes=False)
if "use_tc_tiling_on_sc" in pltpu.CompilerParams.__dataclass_fields__:
  _sc_params = dataclasses.replace(_sc_params, use_tc_tiling_on_sc=False)


def _make_sc_deg(n_chunks):
  scratch = [
      pltpu.VMEM((n_chunks, CH), jnp.int32),
      pltpu.VMEM((NDR, D), jnp.float32),
      pltpu.VMEM((1, NDR), jnp.int32),
      pltpu.VMEM_SHARED((NDR, D), jnp.float32),
  ]

  def body(dstr_hbm, deg_hbm, dst_v, deg_v, ridx_v, deg_sh):
    cid = lax.axis_index("c")
    sid = lax.axis_index("s")
    wid = cid * 16 + sid

    zero16 = jnp.zeros((16,), jnp.float32)

    @pl.loop(0, NDR)
    def _(r):
      @pl.loop(0, D, step=16)
      def _(c):
        deg_v[r, pl.ds(c, 16)] = zero16

    @pl.when(sid == 0)
    def _():
      pltpu.sync_copy(deg_v, deg_sh)
    iota16 = lax.iota(jnp.int32, 16)
    for j in range(NDR // 16):
      ridx_v[0, pl.ds(j * 16, 16)] = iota16 + j * 16

    plsc.subcore_barrier()

    pltpu.sync_copy(dstr_hbm.at[wid], dst_v)
    ones16 = jnp.ones((16,), jnp.float32)

    @pl.loop(0, n_chunks)
    def _(i):
      for j in range(CH // 16):
        idx = dst_v[i, pl.ds(j * 16, 16)]
        plsc.addupdate_scatter(
            deg_v,
            [lax.shift_right_logical(idx, 7), lax.bitwise_and(idx, 127)],
            ones16)

    pltpu.sync_copy(deg_v, deg_sh.at[ridx_v.at[0]], add=True)
    plsc.subcore_barrier()

    @pl.when(sid == 0)
    def _():
      pltpu.sync_copy(deg_sh, deg_hbm.at[pl.ds(cid * NDR, NDR)])

  return pl.kernel(
      body,
      out_type=jax.ShapeDtypeStruct((2 * NDR, D), jnp.float32),
      mesh=_mesh, scratch_types=scratch, compiler_params=_sc_params)


def _make_sc_agg(n_chunks):
  nct = 2 * n_chunks
  assert nct % NBUF == 0
  scratch = [
      pltpu.VMEM((nct, CH), jnp.int32),
      pltpu.VMEM((nct, CH), jnp.int32),
  ]
  scratch += [pltpu.VMEM((CH, DH), jnp.float32) for _ in range(NBUF)]
  scratch += [
      pltpu.VMEM_SHARED((N_PAD, DH), jnp.float32),
  ]
  scratch += [pltpu.SemaphoreType.DMA for _ in range(2 * NBUF)]

  def body(hlo_hbm, hhi_hbm, srcr_hbm, dstr_hbm, alo_hbm, ahi_hbm,
           src_v, dst_v, *rest):
    bufs = rest[:NBUF]
    acc_sh = rest[NBUF]
    gsems = rest[NBUF + 1:NBUF + 1 + NBUF]
    ssems = rest[NBUF + 1 + NBUF:]
    cid = lax.axis_index("c")
    sid = lax.axis_index("s")

    zero16 = jnp.zeros((16,), jnp.float32)

    @pl.loop(0, CH)
    def _(r):
      @pl.loop(0, DH, step=16)
      def _(c):
        bufs[0][r, pl.ds(c, 16)] = zero16

    for k in range(ROWS_PER_TILE // CH):
      pltpu.sync_copy(bufs[0],
                      acc_sh.at[pl.ds(sid * ROWS_PER_TILE + k * CH, CH)])

    plsc.subcore_barrier()

    pltpu.sync_copy(srcr_hbm.at[2 * sid], src_v.at[pl.ds(0, n_chunks)])
    pltpu.sync_copy(srcr_hbm.at[2 * sid + 1], src_v.at[pl.ds(n_chunks, n_chunks)])
    pltpu.sync_copy(dstr_hbm.at[2 * sid], dst_v.at[pl.ds(0, n_chunks)])
    pltpu.sync_copy(dstr_hbm.at[2 * sid + 1], dst_v.at[pl.ds(n_chunks, n_chunks)])

    def pipeline(h_hbm):
      def g_start(i, k):
        pltpu.async_copy(h_hbm.at[src_v.at[i]], bufs[k], gsems[k])

      def g_wait(i, k):
        pltpu.make_async_copy(h_hbm.at[src_v.at[i]], bufs[k], gsems[k]).wait()

      def s_start(i, k):
        pltpu.async_copy(bufs[k], acc_sh.at[dst_v.at[i]], ssems[k], add=True)

      def s_wait(i, k):
        pltpu.make_async_copy(bufs[k], acc_sh.at[dst_v.at[i]], ssems[k]).wait()

      @pl.loop(0, nct, step=NBUF)
      def _(i):
        for k in range(NBUF):
          g_start(i + k, k)
        for k in range(NBUF):
          g_wait(i + k, k)
          s_start(i + k, k)
        for k in range(NBUF):
          s_wait(i + k, k)

    @pl.when(cid == 0)
    def _():
      pipeline(hlo_hbm)

    @pl.when(cid == 1)
    def _():
      pipeline(hhi_hbm)

    plsc.subcore_barrier()

    base = sid * ROWS_PER_TILE

    @pl.when(cid == 0)
    def _():
      pltpu.sync_copy(acc_sh.at[pl.ds(base, ROWS_PER_TILE)],
                      alo_hbm.at[pl.ds(base, ROWS_PER_TILE)])

    @pl.when(cid == 1)
    def _():
      pltpu.sync_copy(acc_sh.at[pl.ds(base, ROWS_PER_TILE)],
                      ahi_hbm.at[pl.ds(base, ROWS_PER_TILE)])

  half = jax.ShapeDtypeStruct((N_PAD, DH), jnp.float32)
  return pl.kernel(
      body,
      out_type=[half, half],
      mesh=_mesh, scratch_types=scratch, compiler_params=_sc_params)


_DOT = functools.partial(
    lax.dot_general,
    dimension_numbers=(((1,), (0,)), ((), ())),
    preferred_element_type=jnp.float32,
    precision=lax.Precision.HIGHEST)


def _k_in_body(x_ref, w_ref, b_ref, lo_ref, hi_ref):
  t = jnp.tanh(_DOT(x_ref[...], w_ref[...]) + b_ref[...])
  lo_ref[...] = t[:, :DH]
  hi_ref[...] = t[:, DH:]


def _layer_math(lo_ref, hi_ref, alo_ref, ahi_ref, d0_ref, d1_ref, ws_ref,
                b_ref, wn_ref):
  deg = jnp.maximum(d0_ref[...] + d1_ref[...], 1.0)
  ws = ws_ref[...]
  wn = wn_ref[...]
  t = _DOT(lo_ref[...], ws[:DH]) + _DOT(hi_ref[...], ws[DH:])
  t += _DOT(alo_ref[...] / deg, wn[:DH]) + _DOT(ahi_ref[...] / deg, wn[DH:])
  return jnp.maximum(t + b_ref[...], 0.0)


def _k_layer_body(lo_ref, hi_ref, alo_ref, ahi_ref, d0_ref, d1_ref, ws_ref,
                  b_ref, wn_ref, olo_ref, ohi_ref):
  r = _layer_math(lo_ref, hi_ref, alo_ref, ahi_ref, d0_ref, d1_ref, ws_ref,
                  b_ref, wn_ref)
  olo_ref[...] = r[:, :DH]
  ohi_ref[...] = r[:, DH:]


def _k_last_body(lo_ref, hi_ref, alo_ref, ahi_ref, d0_ref, d1_ref, ws_ref,
                 b_ref, wn_ref, o_ref):
  o_ref[...] = _layer_math(lo_ref, hi_ref, alo_ref, ahi_ref, d0_ref, d1_ref,
                           ws_ref, b_ref, wn_ref)


_BLK = 1024
_GRID = N_PAD // _BLK
_row_spec = pl.BlockSpec((_BLK, D), lambda i: (i, 0))
_half_spec = pl.BlockSpec((_BLK, DH), lambda i: (i, 0))
_w_spec = pl.BlockSpec((D, D), lambda i: (0, 0))
_b_spec = pl.BlockSpec((1, D), lambda i: (0, 0))
_half_sds = jax.ShapeDtypeStruct((N_PAD, DH), jnp.float32)

_k_in = pl.pallas_call(
    _k_in_body,
    grid=(_GRID,),
    in_specs=[_row_spec, _w_spec, _b_spec],
    out_specs=[_half_spec, _half_spec],
    out_shape=[_half_sds, _half_sds])

_layer_in_specs = [
    _half_spec, _half_spec,
    _half_spec, _half_spec,
    pl.BlockSpec((_BLK, 1), lambda i: (i, 0)),
    pl.BlockSpec((_BLK, 1), lambda i: (i + _GRID, 0)),
    _w_spec, _b_spec, _w_spec,
]

_k_layer = pl.pallas_call(
    _k_layer_body,
    grid=(_GRID,),
    in_specs=_layer_in_specs,
    out_specs=[_half_spec, _half_spec],
    out_shape=[_half_sds, _half_sds])

_k_last = pl.pallas_call(
    _k_last_body,
    grid=(_GRID,),
    in_specs=_layer_in_specs,
    out_specs=_row_spec,
    out_shape=jax.ShapeDtypeStruct((N_PAD, D), jnp.float32))


def kernel(x, edge_index, W_in, b_in, W_self1, b_self1, W_neigh1,
           W_self2, b_self2, W_neigh2):
  E = edge_index.shape[1]
  n_chunks = -(-E // (NW * CH))
  if n_chunks % 2:
    n_chunks += 1
  e_pad = NW * CH * n_chunks - E

  xp = jnp.zeros((N_PAD, D), jnp.float32).at[:N].set(x)
  src = edge_index[0]
  dst = edge_index[1]
  if e_pad:
    ar = jnp.arange(e_pad, dtype=jnp.int32)
    src = jnp.concatenate([src, ar % N])
    dst = jnp.concatenate([dst, N + ar % (N_PAD - N)])
  srcr = src.reshape(NW, n_chunks, CH)
  dstr = dst.reshape(NW, n_chunks, CH)
  srcr, dstr = lax.optimization_barrier((srcr, dstr))

  sc_deg = _make_sc_deg(n_chunks)
  sc_agg = _make_sc_agg(n_chunks)

  b_in2 = b_in.reshape(1, D)
  b1 = b_self1.reshape(1, D)
  b2 = b_self2.reshape(1, D)

  deg = sc_deg(dstr)
  degf = deg.reshape(2 * N_PAD, 1)
  lo0, hi0 = _k_in(xp, W_in, b_in2)
  alo1, ahi1 = sc_agg(lo0, hi0, srcr, dstr)
  lo1, hi1 = _k_layer(lo0, hi0, alo1, ahi1, degf, degf, W_self1, b1, W_neigh1)
  alo2, ahi2 = sc_agg(lo1, hi1, srcr, dstr)
  h2 = _k_last(lo1, hi1, alo2, ahi2, degf, degf, W_self2, b2, W_neigh2)
  return h2[:N]
